# Initial kernel scaffold; baseline (speedup 1.0000x reference)
import functools

import jax
import jax.numpy as jnp
from jax import lax
from jax.experimental import pallas as pl
from jax.experimental.pallas import tpu as pltpu
from jax.experimental.pallas import tpu_sc as plsc

N_NODES = 10000
N_EDGES = 320000
D = 128
N_REL = 4
MAX_DEGREE = 10
N_GRAPHS = 64
N_PAD = 10240
BN = 2048
GRID_N = N_PAD // BN


def _embed_body(x_ref, emb_ref, out_ref):
    x = x_ref[...]
    m = jnp.max(x, axis=1, keepdims=True)
    col = lax.broadcasted_iota(jnp.int32, x.shape, 1)
    am = jnp.min(jnp.where(x == m, col, D), axis=1, keepdims=True)
    onehot = (col == am).astype(jnp.float32)
    out_ref[...] = jnp.dot(onehot, jnp.maximum(emb_ref[...], 0.0),
                           preferred_element_type=jnp.float32)


def _embed(x_pad, emb):
    return pl.pallas_call(
        _embed_body,
        grid=(GRID_N,),
        in_specs=[pl.BlockSpec((BN, D), lambda i: (i, 0)),
                  pl.BlockSpec((D, D), lambda i: (0, 0))],
        out_specs=pl.BlockSpec((BN, D), lambda i: (i, 0)),
        out_shape=jax.ShapeDtypeStruct((N_PAD, D), jnp.float32),
    )(x_pad, emb)


def _rgcn_body(hr_ref, agg_ref, cnt_ref, root_ref, w_ref, b_ref, out_ref):
    hr = hr_ref[...]
    out = jnp.dot(hr, root_ref[...], preferred_element_type=jnp.float32)
    out = out + b_ref[...]
    for r in range(N_REL):
        cnt = jnp.maximum(cnt_ref[r, :], 1.0)[:, None]
        mean = agg_ref[r, :, :] / cnt
        out = out + jnp.dot(mean, w_ref[r, :, :],
                            preferred_element_type=jnp.float32)
    out_ref[...] = jnp.maximum(out, 0.0)


def _rgcn_update(hr, agg, cnt8, root, w, b):
    return pl.pallas_call(
        _rgcn_body,
        grid=(GRID_N,),
        in_specs=[pl.BlockSpec((BN, D), lambda i: (i, 0)),
                  pl.BlockSpec((N_REL, BN, D), lambda i: (0, i, 0)),
                  pl.BlockSpec((8, BN), lambda i: (0, i)),
                  pl.BlockSpec((D, D), lambda i: (0, 0)),
                  pl.BlockSpec((N_REL, D, D), lambda i: (0, 0, 0)),
                  pl.BlockSpec((1, D), lambda i: (0, 0))],
        out_specs=pl.BlockSpec((BN, D), lambda i: (i, 0)),
        out_shape=jax.ShapeDtypeStruct((N_PAD, D), jnp.float32),
    )(hr, agg, cnt8, root, w, b)


def _mf_body(apply_relu, agg_ref, h_ref, cnt_ref, lw_ref, lb_ref, rw_ref,
             out_ref):
    agg = agg_ref[...]
    h = h_ref[...]
    deg = cnt_ref[0, :] + cnt_ref[1, :] + cnt_ref[2, :] + cnt_ref[3, :]
    deg = jnp.clip(deg, 0.0, float(MAX_DEGREE))
    out = jnp.zeros_like(h)
    for k in range(MAX_DEGREE + 1):
        t = (jnp.dot(agg, lw_ref[k, :, :], preferred_element_type=jnp.float32)
             + lb_ref[k:k + 1, :]
             + jnp.dot(h, rw_ref[k, :, :], preferred_element_type=jnp.float32))
        out = out + jnp.where((deg == float(k))[:, None], t, 0.0)
    if apply_relu:
        out = jnp.maximum(out, 0.0)
    out_ref[...] = out


def _mf_update(agg2, h2, cnt8, lw, lb, rw, apply_relu):
    K = MAX_DEGREE + 1
    return pl.pallas_call(
        functools.partial(_mf_body, apply_relu),
        grid=(GRID_N,),
        in_specs=[pl.BlockSpec((BN, D), lambda i: (i, 0)),
                  pl.BlockSpec((BN, D), lambda i: (i, 0)),
                  pl.BlockSpec((8, BN), lambda i: (0, i)),
                  pl.BlockSpec((K, D, D), lambda i: (0, 0, 0)),
                  pl.BlockSpec((K, D), lambda i: (0, 0)),
                  pl.BlockSpec((K, D, D), lambda i: (0, 0, 0))],
        out_specs=pl.BlockSpec((BN, D), lambda i: (i, 0)),
        out_shape=jax.ShapeDtypeStruct((N_PAD, D), jnp.float32),
    )(agg2, h2, cnt8, lw, lb, rw)


def _pool_body(h_ref, bi_ref, w1_ref, b1_ref, w2_ref, b2_ref, out_ref):
    bi = bi_ref[...]
    g = lax.broadcasted_iota(jnp.int32, bi.shape, 1)
    oh = (bi == g).astype(jnp.float32)
    pooled = lax.dot_general(oh, h_ref[...], (((0,), (0,)), ((), ())),
                             preferred_element_type=jnp.float32)
    pooled = pooled[:N_GRAPHS, :]
    t = jnp.maximum(jnp.dot(pooled, w1_ref[...],
                            preferred_element_type=jnp.float32)
                    + b1_ref[...], 0.0)
    out_ref[...] = jnp.dot(t, w2_ref[...],
                           preferred_element_type=jnp.float32) + b2_ref[...]


def _pool_mlp(hf, bi_b, w1, b1, w2, b2):
    return pl.pallas_call(
        _pool_body,
        in_specs=[pl.BlockSpec((N_PAD, D), lambda: (0, 0)),
                  pl.BlockSpec((N_PAD, D), lambda: (0, 0)),
                  pl.BlockSpec((D, D), lambda: (0, 0)),
                  pl.BlockSpec((1, D), lambda: (0, 0)),
                  pl.BlockSpec((D, D), lambda: (0, 0)),
                  pl.BlockSpec((1, D), lambda: (0, 0))],
        out_specs=pl.BlockSpec((N_GRAPHS, D), lambda: (0, 0)),
        out_shape=jax.ShapeDtypeStruct((N_GRAPHS, D), jnp.float32),
    )(hf, bi_b, w1, b1, w2, b2)


def _agg_rel_jnp(hr, src, dst, edge_type):
    h_src = jnp.take(hr[:N_NODES], src, axis=0)
    aggs, cnts = [], []
    for r in range(N_REL):
        mask = (edge_type == r).astype(jnp.float32)
        aggs.append(jax.ops.segment_sum(h_src * mask[:, None], dst,
                                        num_segments=N_NODES))
        cnts.append(jax.ops.segment_sum(mask, dst, num_segments=N_NODES))
    agg = jnp.pad(jnp.stack(aggs), ((0, 0), (0, N_PAD - N_NODES), (0, 0)))
    cnt = jnp.pad(jnp.stack(cnts), ((0, 4), (0, N_PAD - N_NODES)))
    return agg, cnt


def _agg_all_jnp(h2, src, dst):
    agg = jax.ops.segment_sum(jnp.take(h2[:N_NODES], src, axis=0), dst,
                              num_segments=N_NODES)
    return jnp.pad(agg, ((0, N_PAD - N_NODES), (0, 0)))


def kernel(x, edge_index, edge_attr, batch_idx, emb, rgcn_w, rgcn_root,
           rgcn_b, mf_lin_w, mf_lin_b, mf_root_w, lin1_w, lin1_b, lin2_w,
           lin2_b):
    src = edge_index[0]
    dst = edge_index[1]
    edge_type = jnp.argmax(edge_attr, axis=-1).astype(jnp.int32)

    x_pad = jnp.pad(x, ((0, N_PAD - N_NODES), (0, 0)))
    bi_b = jnp.broadcast_to(
        jnp.pad(batch_idx, (0, N_PAD - N_NODES), constant_values=120)[:, None],
        (N_PAD, D))

    h = _embed(x_pad, emb)

    cnt8 = None
    for b in range(2):
        agg, cnt_new = _agg_rel_jnp(h, src, dst, edge_type)
        if cnt8 is None:
            cnt8 = cnt_new
        h2 = _rgcn_update(h, agg, cnt8, rgcn_root[b], rgcn_w[b],
                          rgcn_b[b][None, :])
        agg2 = _agg_all_jnp(h2, src, dst)
        h = _mf_update(agg2, h2, cnt8, mf_lin_w[b], mf_lin_b[b],
                       mf_root_w[b], apply_relu=(b == 0))

    return _pool_mlp(h, bi_b, lin1_w, lin1_b[None, :], lin2_w, lin2_b[None, :])


# speedup vs baseline: 1.0097x; 1.0097x over previous
import functools

import jax
import jax.numpy as jnp
from jax import lax
from jax.experimental import pallas as pl
from jax.experimental.pallas import tpu as pltpu
from jax.experimental.pallas import tpu_sc as plsc

N_NODES = 10000
N_EDGES = 320000
D = 128
N_REL = 4
MAX_DEGREE = 10
N_GRAPHS = 64
N_PAD = 10240
BN = 2048
GRID_N = N_PAD // BN


def _embed_body(x_ref, emb_ref, out_ref):
    x = x_ref[...]
    m = jnp.max(x, axis=1, keepdims=True)
    col = lax.broadcasted_iota(jnp.int32, x.shape, 1)
    am = jnp.min(jnp.where(x == m, col, D), axis=1, keepdims=True)
    onehot = (col == am).astype(jnp.float32)
    out_ref[...] = jnp.dot(onehot, jnp.maximum(emb_ref[...], 0.0),
                           preferred_element_type=jnp.float32)


def _embed(x_pad, emb):
    return pl.pallas_call(
        _embed_body,
        grid=(GRID_N,),
        in_specs=[pl.BlockSpec((BN, D), lambda i: (i, 0)),
                  pl.BlockSpec((D, D), lambda i: (0, 0))],
        out_specs=pl.BlockSpec((BN, D), lambda i: (i, 0)),
        out_shape=jax.ShapeDtypeStruct((N_PAD, D), jnp.float32),
    )(x_pad, emb)


def _rgcn_body(hr_ref, agg_ref, cnt_ref, root_ref, w_ref, b_ref, out_ref):
    hr = hr_ref[...]
    out = jnp.dot(hr, root_ref[...], preferred_element_type=jnp.float32)
    out = out + b_ref[...]
    for r in range(N_REL):
        cnt = jnp.maximum(cnt_ref[:, r:r + 1], 1.0)
        mean = agg_ref[r, :, :] / cnt
        out = out + jnp.dot(mean, w_ref[r, :, :],
                            preferred_element_type=jnp.float32)
    out_ref[...] = jnp.maximum(out, 0.0)


def _rgcn_update(hr, agg, cnt_w, root, w, b):
    return pl.pallas_call(
        _rgcn_body,
        grid=(GRID_N,),
        in_specs=[pl.BlockSpec((BN, D), lambda i: (i, 0)),
                  pl.BlockSpec((N_REL, BN, D), lambda i: (0, i, 0)),
                  pl.BlockSpec((BN, D), lambda i: (i, 0)),
                  pl.BlockSpec((D, D), lambda i: (0, 0)),
                  pl.BlockSpec((N_REL, D, D), lambda i: (0, 0, 0)),
                  pl.BlockSpec((1, D), lambda i: (0, 0))],
        out_specs=pl.BlockSpec((BN, D), lambda i: (i, 0)),
        out_shape=jax.ShapeDtypeStruct((N_PAD, D), jnp.float32),
    )(hr, agg, cnt_w, root, w, b)


def _mf_body(apply_relu, agg_ref, h_ref, cnt_ref, lw_ref, lb_ref, rw_ref,
             out_ref):
    agg = agg_ref[...]
    h = h_ref[...]
    deg = (cnt_ref[:, 0:1] + cnt_ref[:, 1:2] + cnt_ref[:, 2:3]
           + cnt_ref[:, 3:4])
    deg = jnp.clip(deg, 0.0, float(MAX_DEGREE))
    out = jnp.zeros_like(h)
    for k in range(MAX_DEGREE + 1):
        t = (jnp.dot(agg, lw_ref[k, :, :], preferred_element_type=jnp.float32)
             + lb_ref[k:k + 1, :]
             + jnp.dot(h, rw_ref[k, :, :], preferred_element_type=jnp.float32))
        out = out + jnp.where(deg == float(k), t, 0.0)
    if apply_relu:
        out = jnp.maximum(out, 0.0)
    out_ref[...] = out


def _mf_update(agg2, h2, cnt_w, lw, lb, rw, apply_relu):
    K = MAX_DEGREE + 1
    return pl.pallas_call(
        functools.partial(_mf_body, apply_relu),
        grid=(GRID_N,),
        in_specs=[pl.BlockSpec((BN, D), lambda i: (i, 0)),
                  pl.BlockSpec((BN, D), lambda i: (i, 0)),
                  pl.BlockSpec((BN, D), lambda i: (i, 0)),
                  pl.BlockSpec((K, D, D), lambda i: (0, 0, 0)),
                  pl.BlockSpec((K, D), lambda i: (0, 0)),
                  pl.BlockSpec((K, D, D), lambda i: (0, 0, 0))],
        out_specs=pl.BlockSpec((BN, D), lambda i: (i, 0)),
        out_shape=jax.ShapeDtypeStruct((N_PAD, D), jnp.float32),
    )(agg2, h2, cnt_w, lw, lb, rw)


def _pool_body(h_ref, bi_ref, w1_ref, b1_ref, w2_ref, b2_ref, out_ref):
    bi = bi_ref[...]
    g = lax.broadcasted_iota(jnp.int32, bi.shape, 1)
    oh = (bi == g).astype(jnp.float32)
    pooled = lax.dot_general(oh, h_ref[...], (((0,), (0,)), ((), ())),
                             preferred_element_type=jnp.float32)
    pooled = pooled[:N_GRAPHS, :]
    t = jnp.maximum(jnp.dot(pooled, w1_ref[...],
                            preferred_element_type=jnp.float32)
                    + b1_ref[...], 0.0)
    out_ref[...] = jnp.dot(t, w2_ref[...],
                           preferred_element_type=jnp.float32) + b2_ref[...]


def _pool_mlp(hf, bi_b, w1, b1, w2, b2):
    return pl.pallas_call(
        _pool_body,
        in_specs=[pl.BlockSpec((N_PAD, D), lambda: (0, 0)),
                  pl.BlockSpec((N_PAD, D), lambda: (0, 0)),
                  pl.BlockSpec((D, D), lambda: (0, 0)),
                  pl.BlockSpec((1, D), lambda: (0, 0)),
                  pl.BlockSpec((D, D), lambda: (0, 0)),
                  pl.BlockSpec((1, D), lambda: (0, 0))],
        out_specs=pl.BlockSpec((N_GRAPHS, D), lambda: (0, 0)),
        out_shape=jax.ShapeDtypeStruct((N_GRAPHS, D), jnp.float32),
    )(hf, bi_b, w1, b1, w2, b2)


def _agg_rel_jnp(hr, src, dst, edge_type):
    h_src = jnp.take(hr[:N_NODES], src, axis=0)
    aggs, cnts = [], []
    for r in range(N_REL):
        mask = (edge_type == r).astype(jnp.float32)
        aggs.append(jax.ops.segment_sum(h_src * mask[:, None], dst,
                                        num_segments=N_NODES))
        cnts.append(jax.ops.segment_sum(mask, dst, num_segments=N_NODES))
    agg = jnp.pad(jnp.stack(aggs), ((0, 0), (0, N_PAD - N_NODES), (0, 0)))
    cnt_w = jnp.pad(jnp.stack(cnts, axis=1),
                    ((0, N_PAD - N_NODES), (0, D - N_REL)))
    return agg, cnt_w


def _agg_all_jnp(h2, src, dst):
    agg = jax.ops.segment_sum(jnp.take(h2[:N_NODES], src, axis=0), dst,
                              num_segments=N_NODES)
    return jnp.pad(agg, ((0, N_PAD - N_NODES), (0, 0)))


def kernel(x, edge_index, edge_attr, batch_idx, emb, rgcn_w, rgcn_root,
           rgcn_b, mf_lin_w, mf_lin_b, mf_root_w, lin1_w, lin1_b, lin2_w,
           lin2_b):
    src = edge_index[0]
    dst = edge_index[1]
    edge_type = jnp.argmax(edge_attr, axis=-1).astype(jnp.int32)

    x_pad = jnp.pad(x, ((0, N_PAD - N_NODES), (0, 0)))
    bi_b = jnp.broadcast_to(
        jnp.pad(batch_idx, (0, N_PAD - N_NODES), constant_values=120)[:, None],
        (N_PAD, D))

    h = _embed(x_pad, emb)

    cnt_w = None
    for b in range(2):
        agg, cnt_new = _agg_rel_jnp(h, src, dst, edge_type)
        if cnt_w is None:
            cnt_w = cnt_new
        h2 = _rgcn_update(h, agg, cnt_w, rgcn_root[b], rgcn_w[b],
                          rgcn_b[b][None, :])
        agg2 = _agg_all_jnp(h2, src, dst)
        h = _mf_update(agg2, h2, cnt_w, mf_lin_w[b], mf_lin_b[b],
                       mf_root_w[b], apply_relu=(b == 0))

    return _pool_mlp(h, bi_b, lin1_w, lin1_b[None, :], lin2_w, lin2_b[None, :])


# speedup vs baseline: 2.2286x; 2.2072x over previous
import functools

import jax
import jax.numpy as jnp
from jax import lax
from jax.experimental import pallas as pl
from jax.experimental.pallas import tpu as pltpu
from jax.experimental.pallas import tpu_sc as plsc

N_NODES = 10000
N_EDGES = 320000
D = 128
N_REL = 4
MAX_DEGREE = 10
N_GRAPHS = 64
N_PAD = 10240
BN = 2048
GRID_N = N_PAD // BN


def _embed_body(x_ref, emb_ref, out_ref):
    x = x_ref[...]
    m = jnp.max(x, axis=1, keepdims=True)
    col = lax.broadcasted_iota(jnp.int32, x.shape, 1)
    am = jnp.min(jnp.where(x == m, col, D), axis=1, keepdims=True)
    onehot = (col == am).astype(jnp.float32)
    out_ref[...] = jnp.dot(onehot, jnp.maximum(emb_ref[...], 0.0),
                           preferred_element_type=jnp.float32)


def _embed(x_pad, emb):
    return pl.pallas_call(
        _embed_body,
        grid=(GRID_N,),
        in_specs=[pl.BlockSpec((BN, D), lambda i: (i, 0)),
                  pl.BlockSpec((D, D), lambda i: (0, 0))],
        out_specs=pl.BlockSpec((BN, D), lambda i: (i, 0)),
        out_shape=jax.ShapeDtypeStruct((N_PAD, D), jnp.float32),
    )(x_pad, emb)


def _rgcn_body(hr_ref, agg_ref, cnt_ref, root_ref, w_ref, b_ref, out_ref):
    hr = hr_ref[...]
    out = jnp.dot(hr, root_ref[...], preferred_element_type=jnp.float32)
    out = out + b_ref[...]
    for r in range(N_REL):
        cnt = jnp.maximum(cnt_ref[:, r:r + 1], 1.0)
        mean = agg_ref[r, :, :] / cnt
        out = out + jnp.dot(mean, w_ref[r, :, :],
                            preferred_element_type=jnp.float32)
    out_ref[...] = jnp.maximum(out, 0.0)


def _rgcn_update(hr, agg, cnt_w, root, w, b):
    return pl.pallas_call(
        _rgcn_body,
        grid=(GRID_N,),
        in_specs=[pl.BlockSpec((BN, D), lambda i: (i, 0)),
                  pl.BlockSpec((N_REL, BN, D), lambda i: (0, i, 0)),
                  pl.BlockSpec((BN, D), lambda i: (i, 0)),
                  pl.BlockSpec((D, D), lambda i: (0, 0)),
                  pl.BlockSpec((N_REL, D, D), lambda i: (0, 0, 0)),
                  pl.BlockSpec((1, D), lambda i: (0, 0))],
        out_specs=pl.BlockSpec((BN, D), lambda i: (i, 0)),
        out_shape=jax.ShapeDtypeStruct((N_PAD, D), jnp.float32),
    )(hr, agg, cnt_w, root, w, b)


def _mf_body(apply_relu, agg_ref, h_ref, cnt_ref, lw_ref, lb_ref, rw_ref,
             out_ref):
    agg = agg_ref[...]
    h = h_ref[...]
    deg = (cnt_ref[:, 0:1] + cnt_ref[:, 1:2] + cnt_ref[:, 2:3]
           + cnt_ref[:, 3:4])
    deg = jnp.clip(deg, 0.0, float(MAX_DEGREE))
    out = jnp.zeros_like(h)
    for k in range(MAX_DEGREE + 1):
        t = (jnp.dot(agg, lw_ref[k, :, :], preferred_element_type=jnp.float32)
             + lb_ref[k:k + 1, :]
             + jnp.dot(h, rw_ref[k, :, :], preferred_element_type=jnp.float32))
        out = out + jnp.where(deg == float(k), t, 0.0)
    if apply_relu:
        out = jnp.maximum(out, 0.0)
    out_ref[...] = out


def _mf_update(agg2, h2, cnt_w, lw, lb, rw, apply_relu):
    K = MAX_DEGREE + 1
    return pl.pallas_call(
        functools.partial(_mf_body, apply_relu),
        grid=(GRID_N,),
        in_specs=[pl.BlockSpec((BN, D), lambda i: (i, 0)),
                  pl.BlockSpec((BN, D), lambda i: (i, 0)),
                  pl.BlockSpec((BN, D), lambda i: (i, 0)),
                  pl.BlockSpec((K, D, D), lambda i: (0, 0, 0)),
                  pl.BlockSpec((K, D), lambda i: (0, 0)),
                  pl.BlockSpec((K, D, D), lambda i: (0, 0, 0))],
        out_specs=pl.BlockSpec((BN, D), lambda i: (i, 0)),
        out_shape=jax.ShapeDtypeStruct((N_PAD, D), jnp.float32),
    )(agg2, h2, cnt_w, lw, lb, rw)


def _pool_body(h_ref, bi_ref, w1_ref, b1_ref, w2_ref, b2_ref, out_ref):
    bi = bi_ref[...]
    g = lax.broadcasted_iota(jnp.int32, bi.shape, 1)
    oh = (bi == g).astype(jnp.float32)
    pooled = lax.dot_general(oh, h_ref[...], (((0,), (0,)), ((), ())),
                             preferred_element_type=jnp.float32)
    pooled = pooled[:N_GRAPHS, :]
    t = jnp.maximum(jnp.dot(pooled, w1_ref[...],
                            preferred_element_type=jnp.float32)
                    + b1_ref[...], 0.0)
    out_ref[...] = jnp.dot(t, w2_ref[...],
                           preferred_element_type=jnp.float32) + b2_ref[...]


def _pool_mlp(hf, bi_b, w1, b1, w2, b2):
    return pl.pallas_call(
        _pool_body,
        in_specs=[pl.BlockSpec((N_PAD, D), lambda: (0, 0)),
                  pl.BlockSpec((N_PAD, D), lambda: (0, 0)),
                  pl.BlockSpec((D, D), lambda: (0, 0)),
                  pl.BlockSpec((1, D), lambda: (0, 0)),
                  pl.BlockSpec((D, D), lambda: (0, 0)),
                  pl.BlockSpec((1, D), lambda: (0, 0))],
        out_specs=pl.BlockSpec((N_GRAPHS, D), lambda: (0, 0)),
        out_shape=jax.ShapeDtypeStruct((N_GRAPHS, D), jnp.float32),
    )(hf, bi_b, w1, b1, w2, b2)


NC, NS = 2, 16
E_PAD = 327680
BPT = E_PAD // NS // 128
BCH = 16
ECH = BCH * 128
NCHK = BPT // BCH


def _make_sc_agg(nrel, w):
    passes = 2 if nrel == 4 else 1
    chn = N_PAD // (NC * passes)
    accrows = nrel * chn + 8
    trash = nrel * chn
    pt = chn // NS
    nfull = (nrel * chn) // 128

    def body(h_hbm, src_hbm, dst_hbm, typ_hbm, out_hbm, acc,
             rows0, rows1, srcv, dstv, typv, bg0, bg1, bs0, bs1,
             sem0, sem1):
        c = lax.axis_index("c")
        s = lax.axis_index("s")
        rows = (rows0, rows1)
        bg = (bg0, bg1)
        bs = (bs0, bs1)
        sem = (sem0, sem1)

        def zero_rows0():
            def zrow(i, carry):
                for j in range(w // 16):
                    rows0[i, pl.ds(j * 16, 16)] = jnp.zeros((16,),
                                                            jnp.float32)
                return carry
            lax.fori_loop(0, 128, zrow, 0)

        def prep(b, qbase):
            sl = b % 2
            for j in range(8):
                o = b * 128 + j * 16
                bg[sl][pl.ds(j * 16, 16)] = srcv[pl.ds(o, 16)]
                d16 = dstv[pl.ds(o, 16)]
                lidx = d16 - qbase
                if nrel > 1:
                    lidx = lidx + typv[pl.ds(o, 16)] * chn
                m = (d16 >= qbase) & (d16 < qbase + chn)
                bs[sl][pl.ds(j * 16, 16)] = jnp.where(m, lidx, trash)

        for p in range(passes):
            qbase = (c * passes + p) * chn
            zero_rows0()
            for k in range((nfull + NS - 1) // NS):
                bi = s + k * NS
                if (k + 1) * NS <= nfull:
                    pltpu.sync_copy(rows0, acc.at[pl.ds(bi * 128, 128)])
                else:
                    @pl.when(bi < nfull)
                    def _():
                        pltpu.sync_copy(rows0,
                                        acc.at[pl.ds(bi * 128, 128)])

            @pl.when(s == 0)
            def _():
                pltpu.sync_copy(rows0.at[pl.ds(0, 8)],
                                acc.at[pl.ds(nfull * 128, 8)])
            plsc.subcore_barrier()

            def chunk(ch, carry):
                ebase = (s * BPT + ch * BCH) * 128
                pltpu.sync_copy(src_hbm.at[pl.ds(ebase, ECH)], srcv)
                pltpu.sync_copy(dst_hbm.at[pl.ds(ebase, ECH)], dstv)
                if nrel > 1:
                    pltpu.sync_copy(typ_hbm.at[pl.ds(ebase, ECH)], typv)
                prep(0, qbase)
                cp0 = pltpu.async_copy(h_hbm.at[bg[0]], rows[0], sem[0])
                for b in range(BCH):
                    sl = b % 2
                    if b == 0:
                        cp = cp0
                    if b + 1 < BCH:
                        prep(b + 1, qbase)
                        nxt = pltpu.async_copy(h_hbm.at[bg[(b + 1) % 2]],
                                               rows[(b + 1) % 2],
                                               sem[(b + 1) % 2])
                    cp.wait()
                    pltpu.sync_copy(rows[sl], acc.at[bs[sl]], add=True)
                    if b + 1 < BCH:
                        cp = nxt
                return carry
            lax.fori_loop(0, NCHK, chunk, 0)

            plsc.subcore_barrier()
            for r in range(nrel):
                pltpu.sync_copy(acc.at[pl.ds(r * chn + s * pt, pt)],
                                out_hbm.at[r, pl.ds(qbase + s * pt, pt)])
            plsc.subcore_barrier()

    return pl.kernel(
        body,
        out_type=jax.ShapeDtypeStruct((nrel, N_PAD, w), jnp.float32),
        mesh=plsc.VectorSubcoreMesh(core_axis_name="c",
                                    subcore_axis_name="s"),
        scratch_types=[
            pltpu.VMEM_SHARED((accrows, w), jnp.float32),
            pltpu.VMEM((128, w), jnp.float32),
            pltpu.VMEM((128, w), jnp.float32),
            pltpu.VMEM((ECH,), jnp.int32),
            pltpu.VMEM((ECH,), jnp.int32),
            pltpu.VMEM((ECH,), jnp.int32),
            pltpu.VMEM((128,), jnp.int32),
            pltpu.VMEM((128,), jnp.int32),
            pltpu.VMEM((128,), jnp.int32),
            pltpu.VMEM((128,), jnp.int32),
            pltpu.SemaphoreType.DMA,
            pltpu.SemaphoreType.DMA,
        ],
    )


_sc_agg4 = _make_sc_agg(4, D)
_sc_agg1 = _make_sc_agg(1, D)

CROWS = N_REL * (N_PAD // 128)
CACC = 328
CTRASH = CROWS
BPT_C = E_PAD // (NC * NS) // 128
NCHK_C = BPT_C // BCH


def _make_sc_cnt():
    def body(ident_hbm, dst_hbm, typ_hbm, out_hbm, acc,
             rows0, rows1, dstv, typv, bg0, bg1, bs0, bs1, sem0, sem1):
        c = lax.axis_index("c")
        s = lax.axis_index("s")
        rows = (rows0, rows1)
        bg = (bg0, bg1)
        bs = (bs0, bs1)
        sem = (sem0, sem1)

        def zrow(i, carry):
            for j in range(D // 16):
                rows0[i, pl.ds(j * 16, 16)] = jnp.zeros((16,), jnp.float32)
            return carry
        lax.fori_loop(0, 128, zrow, 0)

        @pl.when(s < 2)
        def _():
            pltpu.sync_copy(rows0, acc.at[pl.ds(s * 128, 128)])

        @pl.when(s == 2)
        def _():
            pltpu.sync_copy(rows0.at[pl.ds(0, 72)], acc.at[pl.ds(256, 72)])
        plsc.subcore_barrier()

        def prep(b):
            sl = b % 2
            for j in range(8):
                o = b * 128 + j * 16
                d16 = dstv[pl.ds(o, 16)]
                t16 = typv[pl.ds(o, 16)]
                bg[sl][pl.ds(j * 16, 16)] = d16 & 127
                m = d16 < N_NODES
                bs[sl][pl.ds(j * 16, 16)] = jnp.where(
                    m, t16 * (N_PAD // 128) + (d16 >> 7), CTRASH)

        def chunk(ch, carry):
            ebase = ((c * NS + s) * BPT_C + ch * BCH) * 128
            pltpu.sync_copy(dst_hbm.at[pl.ds(ebase, ECH)], dstv)
            pltpu.sync_copy(typ_hbm.at[pl.ds(ebase, ECH)], typv)
            prep(0)
            cp = pltpu.async_copy(ident_hbm.at[bg[0]], rows[0], sem[0])
            for b in range(BCH):
                sl = b % 2
                if b + 1 < BCH:
                    prep(b + 1)
                    nxt = pltpu.async_copy(ident_hbm.at[bg[(b + 1) % 2]],
                                           rows[(b + 1) % 2],
                                           sem[(b + 1) % 2])
                cp.wait()
                pltpu.sync_copy(rows[sl], acc.at[bs[sl]], add=True)
                if b + 1 < BCH:
                    cp = nxt
            return carry
        lax.fori_loop(0, NCHK_C, chunk, 0)

        plsc.subcore_barrier()

        @pl.when(s < 2)
        def _():
            pltpu.sync_copy(acc.at[pl.ds(s * 128, 128)],
                            out_hbm.at[c, pl.ds(s * 128, 128)])

        @pl.when(s == 2)
        def _():
            pltpu.sync_copy(acc.at[pl.ds(256, 72)],
                            out_hbm.at[c, pl.ds(256, 72)])
        plsc.subcore_barrier()

    return pl.kernel(
        body,
        out_type=jax.ShapeDtypeStruct((NC, CACC, D), jnp.float32),
        name="sc_cnt",
        mesh=plsc.VectorSubcoreMesh(core_axis_name="c",
                                    subcore_axis_name="s"),
        scratch_types=[
            pltpu.VMEM_SHARED((CACC, D), jnp.float32),
            pltpu.VMEM((128, D), jnp.float32),
            pltpu.VMEM((128, D), jnp.float32),
            pltpu.VMEM((ECH,), jnp.int32),
            pltpu.VMEM((ECH,), jnp.int32),
            pltpu.VMEM((128,), jnp.int32),
            pltpu.VMEM((128,), jnp.int32),
            pltpu.VMEM((128,), jnp.int32),
            pltpu.VMEM((128,), jnp.int32),
            pltpu.SemaphoreType.DMA,
            pltpu.SemaphoreType.DMA,
        ],
    )


_sc_cnt = _make_sc_cnt()


def _cnt_w_from_sc(dst, edge_type):
    ident = jnp.eye(D, dtype=jnp.float32)
    parts = _sc_cnt(ident, dst, edge_type)
    cnt = (parts[0] + parts[1])[:CROWS].reshape(N_REL, N_PAD).T
    return jnp.pad(cnt, ((0, 0), (0, D - N_REL)))


def kernel(x, edge_index, edge_attr, batch_idx, emb, rgcn_w, rgcn_root,
           rgcn_b, mf_lin_w, mf_lin_b, mf_root_w, lin1_w, lin1_b, lin2_w,
           lin2_b):
    src = jnp.pad(edge_index[0], (0, E_PAD - N_EDGES))
    dst = jnp.pad(edge_index[1], (0, E_PAD - N_EDGES),
                  constant_values=N_PAD)
    edge_type = jnp.pad(jnp.argmax(edge_attr, axis=-1).astype(jnp.int32),
                        (0, E_PAD - N_EDGES))

    x_pad = jnp.pad(x, ((0, N_PAD - N_NODES), (0, 0)))
    bi_b = jnp.broadcast_to(
        jnp.pad(batch_idx, (0, N_PAD - N_NODES), constant_values=120)[:, None],
        (N_PAD, D))

    h = _embed(x_pad, emb)

    cnt_w = _cnt_w_from_sc(dst, edge_type)
    for b in range(2):
        agg4 = _sc_agg4(h, src, dst, edge_type)
        h2 = _rgcn_update(h, agg4, cnt_w, rgcn_root[b],
                          rgcn_w[b], rgcn_b[b][None, :])
        agg2 = _sc_agg1(h2, src, dst, edge_type)[0]
        h = _mf_update(agg2, h2, cnt_w, mf_lin_w[b], mf_lin_b[b],
                       mf_root_w[b], apply_relu=(b == 0))

    return _pool_mlp(h, bi_b, lin1_w, lin1_b[None, :], lin2_w, lin2_b[None, :])


# speedup vs baseline: 2.5426x; 1.1409x over previous
import functools

import jax
import jax.numpy as jnp
from jax import lax
from jax.experimental import pallas as pl
from jax.experimental.pallas import tpu as pltpu
from jax.experimental.pallas import tpu_sc as plsc

N_NODES = 10000
N_EDGES = 320000
D = 128
N_REL = 4
MAX_DEGREE = 10
N_GRAPHS = 64
N_PAD = 10240
BN = 2048
GRID_N = N_PAD // BN


def _embed_body(x_ref, emb_ref, out_ref):
    x = x_ref[...]
    m = jnp.max(x, axis=1, keepdims=True)
    col = lax.broadcasted_iota(jnp.int32, x.shape, 1)
    am = jnp.min(jnp.where(x == m, col, D), axis=1, keepdims=True)
    onehot = (col == am).astype(jnp.float32)
    out_ref[...] = jnp.dot(onehot, jnp.maximum(emb_ref[...], 0.0),
                           preferred_element_type=jnp.float32)


def _embed(x_pad, emb):
    return pl.pallas_call(
        _embed_body,
        grid=(GRID_N,),
        in_specs=[pl.BlockSpec((BN, D), lambda i: (i, 0)),
                  pl.BlockSpec((D, D), lambda i: (0, 0))],
        out_specs=pl.BlockSpec((BN, D), lambda i: (i, 0)),
        out_shape=jax.ShapeDtypeStruct((N_PAD, D), jnp.float32),
    )(x_pad, emb)


def _rgcn_body(hr_ref, agg_ref, cnt_ref, root_ref, w_ref, b_ref, out_ref):
    hr = hr_ref[...]
    out = jnp.dot(hr, root_ref[...], preferred_element_type=jnp.float32)
    out = out + b_ref[...]
    for r in range(N_REL):
        cnt = jnp.maximum(cnt_ref[:, r:r + 1], 1.0)
        mean = agg_ref[r, :, :] / cnt
        out = out + jnp.dot(mean, w_ref[r, :, :],
                            preferred_element_type=jnp.float32)
    out_ref[...] = jnp.maximum(out, 0.0)


def _rgcn_update(hr, agg, cnt_w, root, w, b):
    return pl.pallas_call(
        _rgcn_body,
        grid=(GRID_N,),
        in_specs=[pl.BlockSpec((BN, D), lambda i: (i, 0)),
                  pl.BlockSpec((N_REL, BN, D), lambda i: (0, i, 0)),
                  pl.BlockSpec((BN, D), lambda i: (i, 0)),
                  pl.BlockSpec((D, D), lambda i: (0, 0)),
                  pl.BlockSpec((N_REL, D, D), lambda i: (0, 0, 0)),
                  pl.BlockSpec((1, D), lambda i: (0, 0))],
        out_specs=pl.BlockSpec((BN, D), lambda i: (i, 0)),
        out_shape=jax.ShapeDtypeStruct((N_PAD, D), jnp.float32),
    )(hr, agg, cnt_w, root, w, b)


def _mf_body(apply_relu, agg_ref, h_ref, cnt_ref, lw_ref, lb_ref, rw_ref,
             out_ref):
    agg = agg_ref[...]
    h = h_ref[...]
    deg = (cnt_ref[:, 0:1] + cnt_ref[:, 1:2] + cnt_ref[:, 2:3]
           + cnt_ref[:, 3:4])
    deg = jnp.clip(deg, 0.0, float(MAX_DEGREE))
    out = jnp.zeros_like(h)
    for k in range(MAX_DEGREE + 1):
        t = (jnp.dot(agg, lw_ref[k, :, :], preferred_element_type=jnp.float32)
             + lb_ref[k:k + 1, :]
             + jnp.dot(h, rw_ref[k, :, :], preferred_element_type=jnp.float32))
        out = out + jnp.where(deg == float(k), t, 0.0)
    if apply_relu:
        out = jnp.maximum(out, 0.0)
    out_ref[...] = out


def _mf_update(agg2, h2, cnt_w, lw, lb, rw, apply_relu):
    K = MAX_DEGREE + 1
    return pl.pallas_call(
        functools.partial(_mf_body, apply_relu),
        grid=(GRID_N,),
        in_specs=[pl.BlockSpec((BN, D), lambda i: (i, 0)),
                  pl.BlockSpec((BN, D), lambda i: (i, 0)),
                  pl.BlockSpec((BN, D), lambda i: (i, 0)),
                  pl.BlockSpec((K, D, D), lambda i: (0, 0, 0)),
                  pl.BlockSpec((K, D), lambda i: (0, 0)),
                  pl.BlockSpec((K, D, D), lambda i: (0, 0, 0))],
        out_specs=pl.BlockSpec((BN, D), lambda i: (i, 0)),
        out_shape=jax.ShapeDtypeStruct((N_PAD, D), jnp.float32),
    )(agg2, h2, cnt_w, lw, lb, rw)


def _pool_body(h_ref, bi_ref, w1_ref, b1_ref, w2_ref, b2_ref, out_ref):
    bi = bi_ref[...]
    g = lax.broadcasted_iota(jnp.int32, bi.shape, 1)
    oh = (bi == g).astype(jnp.float32)
    pooled = lax.dot_general(oh, h_ref[...], (((0,), (0,)), ((), ())),
                             preferred_element_type=jnp.float32)
    pooled = pooled[:N_GRAPHS, :]
    t = jnp.maximum(jnp.dot(pooled, w1_ref[...],
                            preferred_element_type=jnp.float32)
                    + b1_ref[...], 0.0)
    out_ref[...] = jnp.dot(t, w2_ref[...],
                           preferred_element_type=jnp.float32) + b2_ref[...]


def _pool_mlp(hf, bi_b, w1, b1, w2, b2):
    return pl.pallas_call(
        _pool_body,
        in_specs=[pl.BlockSpec((N_PAD, D), lambda: (0, 0)),
                  pl.BlockSpec((N_PAD, D), lambda: (0, 0)),
                  pl.BlockSpec((D, D), lambda: (0, 0)),
                  pl.BlockSpec((1, D), lambda: (0, 0)),
                  pl.BlockSpec((D, D), lambda: (0, 0)),
                  pl.BlockSpec((1, D), lambda: (0, 0))],
        out_specs=pl.BlockSpec((N_GRAPHS, D), lambda: (0, 0)),
        out_shape=jax.ShapeDtypeStruct((N_GRAPHS, D), jnp.float32),
    )(hf, bi_b, w1, b1, w2, b2)


NC, NS = 2, 16
E_PAD = 327680
BPT = E_PAD // NS // 128
BCH = 16
ECH = BCH * 128
NCHK = BPT // BCH


def _make_sc_agg(nrel, w):
    split_edges = nrel == 1
    passes = 1 if split_edges else 2
    chn = N_PAD if split_edges else N_PAD // (NC * passes)
    accrows = nrel * chn + 8
    trash = nrel * chn
    pt = chn // NS
    nfull = (nrel * chn) // 128
    bpt = BPT // NC if split_edges else BPT
    nchk = bpt // BCH

    def body(h_hbm, src_hbm, dst_hbm, typ_hbm, out_hbm, acc,
             rows0, rows1, srcv, dstv, typv, bg0, bg1, bs0, bs1,
             sem0, sem1):
        c = lax.axis_index("c")
        s = lax.axis_index("s")
        rows = (rows0, rows1)
        bg = (bg0, bg1)
        bs = (bs0, bs1)
        sem = (sem0, sem1)

        def zero_rows0():
            def zrow(i, carry):
                for j in range(w // 16):
                    rows0[i, pl.ds(j * 16, 16)] = jnp.zeros((16,),
                                                            jnp.float32)
                return carry
            lax.fori_loop(0, 128, zrow, 0)

        def prep(b, qbase):
            sl = b % 2
            for j in range(8):
                o = b * 128 + j * 16
                bg[sl][pl.ds(j * 16, 16)] = srcv[pl.ds(o, 16)]
                d16 = dstv[pl.ds(o, 16)]
                lidx = d16 - qbase
                if nrel > 1:
                    lidx = lidx + typv[pl.ds(o, 16)] * chn
                m = (d16 >= qbase) & (d16 < qbase + chn)
                bs[sl][pl.ds(j * 16, 16)] = jnp.where(m, lidx, trash)

        for p in range(passes):
            qbase = 0 if split_edges else (c * passes + p) * chn
            zero_rows0()
            for k in range((nfull + NS - 1) // NS):
                bi = s + k * NS
                if (k + 1) * NS <= nfull:
                    pltpu.sync_copy(rows0, acc.at[pl.ds(bi * 128, 128)])
                else:
                    @pl.when(bi < nfull)
                    def _():
                        pltpu.sync_copy(rows0,
                                        acc.at[pl.ds(bi * 128, 128)])

            @pl.when(s == 0)
            def _():
                pltpu.sync_copy(rows0.at[pl.ds(0, 8)],
                                acc.at[pl.ds(nfull * 128, 8)])
            plsc.subcore_barrier()

            def chunk(ch, carry):
                if split_edges:
                    ebase = ((c * NS + s) * bpt + ch * BCH) * 128
                else:
                    ebase = (s * bpt + ch * BCH) * 128
                pltpu.sync_copy(src_hbm.at[pl.ds(ebase, ECH)], srcv)
                pltpu.sync_copy(dst_hbm.at[pl.ds(ebase, ECH)], dstv)
                if nrel > 1:
                    pltpu.sync_copy(typ_hbm.at[pl.ds(ebase, ECH)], typv)
                prep(0, qbase)
                cp0 = pltpu.async_copy(h_hbm.at[bg[0]], rows[0], sem[0])
                for b in range(BCH):
                    sl = b % 2
                    if b == 0:
                        cp = cp0
                    if b + 1 < BCH:
                        prep(b + 1, qbase)
                        nxt = pltpu.async_copy(h_hbm.at[bg[(b + 1) % 2]],
                                               rows[(b + 1) % 2],
                                               sem[(b + 1) % 2])
                    cp.wait()
                    pltpu.sync_copy(rows[sl], acc.at[bs[sl]], add=True)
                    if b + 1 < BCH:
                        cp = nxt
                return carry
            lax.fori_loop(0, nchk, chunk, 0)

            plsc.subcore_barrier()
            if split_edges:
                pltpu.sync_copy(acc.at[pl.ds(s * pt, pt)],
                                out_hbm.at[c, pl.ds(s * pt, pt)])
            else:
                for r in range(nrel):
                    pltpu.sync_copy(acc.at[pl.ds(r * chn + s * pt, pt)],
                                    out_hbm.at[r, pl.ds(qbase + s * pt, pt)])
            plsc.subcore_barrier()

    out_leading = NC if split_edges else nrel
    return pl.kernel(
        body,
        out_type=jax.ShapeDtypeStruct((out_leading, N_PAD, w), jnp.float32),
        mesh=plsc.VectorSubcoreMesh(core_axis_name="c",
                                    subcore_axis_name="s"),
        scratch_types=[
            pltpu.VMEM_SHARED((accrows, w), jnp.float32),
            pltpu.VMEM((128, w), jnp.float32),
            pltpu.VMEM((128, w), jnp.float32),
            pltpu.VMEM((ECH,), jnp.int32),
            pltpu.VMEM((ECH,), jnp.int32),
            pltpu.VMEM((ECH,), jnp.int32),
            pltpu.VMEM((128,), jnp.int32),
            pltpu.VMEM((128,), jnp.int32),
            pltpu.VMEM((128,), jnp.int32),
            pltpu.VMEM((128,), jnp.int32),
            pltpu.SemaphoreType.DMA,
            pltpu.SemaphoreType.DMA,
        ],
    )


_sc_agg4 = _make_sc_agg(4, D)
_sc_agg1 = _make_sc_agg(1, D)

CROWS = N_REL * (N_PAD // 128)
CACC = 328
CTRASH = CROWS
BPT_C = E_PAD // (NC * NS) // 128
NCHK_C = BPT_C // BCH


def _make_sc_cnt():
    def body(ident_hbm, dst_hbm, typ_hbm, out_hbm, acc,
             rows0, rows1, dstv, typv, bg0, bg1, bs0, bs1, sem0, sem1):
        c = lax.axis_index("c")
        s = lax.axis_index("s")
        rows = (rows0, rows1)
        bg = (bg0, bg1)
        bs = (bs0, bs1)
        sem = (sem0, sem1)

        def zrow(i, carry):
            for j in range(D // 16):
                rows0[i, pl.ds(j * 16, 16)] = jnp.zeros((16,), jnp.float32)
            return carry
        lax.fori_loop(0, 128, zrow, 0)

        @pl.when(s < 2)
        def _():
            pltpu.sync_copy(rows0, acc.at[pl.ds(s * 128, 128)])

        @pl.when(s == 2)
        def _():
            pltpu.sync_copy(rows0.at[pl.ds(0, 72)], acc.at[pl.ds(256, 72)])
        plsc.subcore_barrier()

        def prep(b):
            sl = b % 2
            for j in range(8):
                o = b * 128 + j * 16
                d16 = dstv[pl.ds(o, 16)]
                t16 = typv[pl.ds(o, 16)]
                bg[sl][pl.ds(j * 16, 16)] = d16 & 127
                m = d16 < N_NODES
                bs[sl][pl.ds(j * 16, 16)] = jnp.where(
                    m, t16 * (N_PAD // 128) + (d16 >> 7), CTRASH)

        def chunk(ch, carry):
            ebase = ((c * NS + s) * BPT_C + ch * BCH) * 128
            pltpu.sync_copy(dst_hbm.at[pl.ds(ebase, ECH)], dstv)
            pltpu.sync_copy(typ_hbm.at[pl.ds(ebase, ECH)], typv)
            prep(0)
            cp = pltpu.async_copy(ident_hbm.at[bg[0]], rows[0], sem[0])
            for b in range(BCH):
                sl = b % 2
                if b + 1 < BCH:
                    prep(b + 1)
                    nxt = pltpu.async_copy(ident_hbm.at[bg[(b + 1) % 2]],
                                           rows[(b + 1) % 2],
                                           sem[(b + 1) % 2])
                cp.wait()
                pltpu.sync_copy(rows[sl], acc.at[bs[sl]], add=True)
                if b + 1 < BCH:
                    cp = nxt
            return carry
        lax.fori_loop(0, NCHK_C, chunk, 0)

        plsc.subcore_barrier()

        @pl.when(s < 2)
        def _():
            pltpu.sync_copy(acc.at[pl.ds(s * 128, 128)],
                            out_hbm.at[c, pl.ds(s * 128, 128)])

        @pl.when(s == 2)
        def _():
            pltpu.sync_copy(acc.at[pl.ds(256, 72)],
                            out_hbm.at[c, pl.ds(256, 72)])
        plsc.subcore_barrier()

    return pl.kernel(
        body,
        out_type=jax.ShapeDtypeStruct((NC, CACC, D), jnp.float32),
        name="sc_cnt",
        mesh=plsc.VectorSubcoreMesh(core_axis_name="c",
                                    subcore_axis_name="s"),
        scratch_types=[
            pltpu.VMEM_SHARED((CACC, D), jnp.float32),
            pltpu.VMEM((128, D), jnp.float32),
            pltpu.VMEM((128, D), jnp.float32),
            pltpu.VMEM((ECH,), jnp.int32),
            pltpu.VMEM((ECH,), jnp.int32),
            pltpu.VMEM((128,), jnp.int32),
            pltpu.VMEM((128,), jnp.int32),
            pltpu.VMEM((128,), jnp.int32),
            pltpu.VMEM((128,), jnp.int32),
            pltpu.SemaphoreType.DMA,
            pltpu.SemaphoreType.DMA,
        ],
    )


_sc_cnt = _make_sc_cnt()


def _cnt_w_from_sc(dst, edge_type):
    ident = jnp.eye(D, dtype=jnp.float32)
    parts = _sc_cnt(ident, dst, edge_type)
    cnt = (parts[0] + parts[1])[:CROWS].reshape(N_REL, N_PAD).T
    return jnp.pad(cnt, ((0, 0), (0, D - N_REL)))


def kernel(x, edge_index, edge_attr, batch_idx, emb, rgcn_w, rgcn_root,
           rgcn_b, mf_lin_w, mf_lin_b, mf_root_w, lin1_w, lin1_b, lin2_w,
           lin2_b):
    src = jnp.pad(edge_index[0], (0, E_PAD - N_EDGES))
    dst = jnp.pad(edge_index[1], (0, E_PAD - N_EDGES),
                  constant_values=N_PAD)
    edge_type = jnp.pad(jnp.argmax(edge_attr, axis=-1).astype(jnp.int32),
                        (0, E_PAD - N_EDGES))

    x_pad = jnp.pad(x, ((0, N_PAD - N_NODES), (0, 0)))
    bi_b = jnp.broadcast_to(
        jnp.pad(batch_idx, (0, N_PAD - N_NODES), constant_values=120)[:, None],
        (N_PAD, D))

    h = _embed(x_pad, emb)

    cnt_w = _cnt_w_from_sc(dst, edge_type)
    for b in range(2):
        agg4 = _sc_agg4(h, src, dst, edge_type)
        h2 = _rgcn_update(h, agg4, cnt_w, rgcn_root[b],
                          rgcn_w[b], rgcn_b[b][None, :])
        parts = _sc_agg1(h2, src, dst, edge_type)
        agg2 = parts[0] + parts[1]
        h = _mf_update(agg2, h2, cnt_w, mf_lin_w[b], mf_lin_b[b],
                       mf_root_w[b], apply_relu=(b == 0))

    return _pool_mlp(h, bi_b, lin1_w, lin1_b[None, :], lin2_w, lin2_b[None, :])
